# Initial kernel scaffold; baseline (speedup 1.0000x reference)
import jax
import jax.numpy as jnp
from jax.experimental import pallas as pl


def kernel(x, edge_index, W1, b1, gamma, beta, W2, b2, Wl1, bl1, Wl2, bl2):
    raise NotImplementedError("write your pallas kernel here")


# speedup vs baseline: 11.8120x; 11.8120x over previous
import jax
import jax.numpy as jnp
from jax import lax
from jax.experimental import pallas as pl
from jax.experimental.pallas import tpu as pltpu
from jax.experimental.pallas import tpu_sc as plsc

N = 10000
E = 320000
D = 128
Q = 32
W = 2 * Q
EPS = 1e-7

NC = 2
NS = 16
C = 80
CHUNKS = E // C
PT = CHUNKS // NS
RW = 10
RT = N // RW


def _table_body(x_ref, t_ref):
    t = jnp.maximum(x_ref[...], 0.0) + EPS
    e = jnp.exp(t)
    n = t * e
    for q in range(4):
        t_ref[q] = jnp.concatenate(
            [e[:, q * Q:(q + 1) * Q], n[:, q * Q:(q + 1) * Q]], axis=1)


def _build_table(x):
    return pl.pallas_call(
        _table_body,
        out_shape=jax.ShapeDtypeStruct((4, N, W), jnp.float32),
    )(x)


def _sc_edge_body(table_hbm, srcq_hbm, dst_hbm, zero_hbm, out_hbm,
                  src_v, dst_v, buf0, buf1, acc, sem0, sem1):
    c = lax.axis_index("c")
    s = lax.axis_index("s")
    pltpu.sync_copy(dst_hbm.at[s], dst_v)

    for p in range(2):
        q = 2 * c + p
        @pl.when(s < RW)
        def _zero():
            pltpu.sync_copy(zero_hbm, acc.at[pl.ds(s * RT, RT)])
        pltpu.sync_copy(srcq_hbm.at[q * NS + s], src_v)
        plsc.subcore_barrier()

        pltpu.async_copy(table_hbm.at[src_v.at[0]], buf0, sem0)

        def body(jj, _):
            j0 = 2 * jj
            j1 = j0 + 1
            pltpu.async_copy(table_hbm.at[src_v.at[j1]], buf1, sem1)
            pltpu.make_async_copy(
                table_hbm.at[src_v.at[j0]], buf0, sem0).wait()
            pltpu.sync_copy(buf0, acc.at[dst_v.at[j0]], add=True)

            @pl.when(jj < PT // 2 - 1)
            def _prefetch():
                pltpu.async_copy(table_hbm.at[src_v.at[j0 + 2]], buf0, sem0)

            pltpu.make_async_copy(
                table_hbm.at[src_v.at[j1]], buf1, sem1).wait()
            pltpu.sync_copy(buf1, acc.at[dst_v.at[j1]], add=True)
            return 0

        lax.fori_loop(0, PT // 2, body, 0)
        plsc.subcore_barrier()
        @pl.when(s < RW)
        def _writeback():
            pltpu.sync_copy(acc.at[pl.ds(s * RT, RT)],
                            out_hbm.at[pl.ds(q * N + s * RT, RT)])
        plsc.subcore_barrier()


_sc_edge = pl.kernel(
    _sc_edge_body,
    out_type=jax.ShapeDtypeStruct((4 * N, W), jnp.float32),
    mesh=plsc.VectorSubcoreMesh(
        core_axis_name="c", subcore_axis_name="s", num_cores=NC,
        num_subcores=NS),
    scratch_types=[
        pltpu.VMEM((PT, C), jnp.int32),
        pltpu.VMEM((PT, C), jnp.int32),
        pltpu.VMEM((C, W), jnp.float32),
        pltpu.VMEM((C, W), jnp.float32),
        pltpu.VMEM_SHARED((N, W), jnp.float32),
        pltpu.SemaphoreType.DMA,
        pltpu.SemaphoreType.DMA,
    ],
    compiler_params=pltpu.CompilerParams(use_tc_tiling_on_sc=False),
)


def _head_body(sums_ref, x_ref, W1_ref, b1_ref, gamma_ref, beta_ref,
               W2_ref, b2_ref, Wl1_ref, bl1_ref, Wl2_ref, bl2_ref, o_ref):
    esum = jnp.concatenate([sums_ref[q, :, :Q] for q in range(4)], axis=1)
    nsum = jnp.concatenate([sums_ref[q, :, Q:] for q in range(4)], axis=1)
    agg = nsum / (esum + 1e-16)
    out = agg + x_ref[...]
    h = jnp.dot(out, W1_ref[...], preferred_element_type=jnp.float32)
    h = h + b1_ref[...]
    mu = jnp.mean(h, axis=0, keepdims=True)
    var = jnp.mean((h - mu) ** 2, axis=0, keepdims=True)
    h = (h - mu) / jnp.sqrt(var + 1e-5) * gamma_ref[...] + beta_ref[...]
    h = jnp.maximum(h, 0.0)
    h = jnp.dot(h, W2_ref[...], preferred_element_type=jnp.float32)
    h = jnp.maximum(h + b2_ref[...], 0.0)
    h = jnp.dot(h, Wl1_ref[...], preferred_element_type=jnp.float32)
    h = jnp.maximum(h + bl1_ref[...], 0.0)
    h = jnp.dot(h, Wl2_ref[...], preferred_element_type=jnp.float32)
    o_ref[...] = jnp.maximum(h + bl2_ref[...], 0.0)


def _head(sums, x, W1, b1, gamma, beta, W2, b2, Wl1, bl1, Wl2, bl2):
    return pl.pallas_call(
        _head_body,
        out_shape=jax.ShapeDtypeStruct((N, D), jnp.float32),
    )(sums, x, W1, b1, gamma, beta, W2, b2, Wl1, bl1, Wl2, bl2)


@jax.jit
def kernel(x, edge_index, W1, b1, gamma, beta, W2, b2, Wl1, bl1, Wl2, bl2):
    table = _build_table(x).reshape(4 * N, W)
    src3d = edge_index[0].reshape(NS, PT, C)
    srcq = jnp.concatenate([src3d + q * N for q in range(4)], axis=0)
    dst3d = edge_index[1].reshape(NS, PT, C)
    zero = jnp.zeros((RT, W), jnp.float32)
    sums = _sc_edge(table, srcq, dst3d, zero)
    sums = sums.reshape(4, N, W)
    return _head(sums, x,
                 W1, b1.reshape(1, -1), gamma.reshape(1, -1),
                 beta.reshape(1, -1), W2, b2.reshape(1, -1),
                 Wl1, bl1.reshape(1, -1), Wl2, bl2.reshape(1, -1))
